# baseline (speedup 1.0000x reference)
import functools

import jax
import jax.numpy as jnp
from jax import lax
from jax.experimental import pallas as pl
from jax.experimental.pallas import tpu as pltpu
from jax.experimental.pallas import tpu_sc as plsc

NNZ_PER_ROW = 7
ROWS_PER_CHUNK = 8
IDX_PER_CHUNK = NNZ_PER_ROW * ROWS_PER_CHUNK
LANES = 16


def _sc_spmm(xt, cols_t, vals_t, iters, n_workers, num_cores, d):
    d2 = d // 2
    d_groups = d2 // LANES
    idx_per_tile = iters * IDX_PER_CHUNK
    w_per_tile = idx_per_tile + LANES
    mesh = plsc.VectorSubcoreMesh(core_axis_name="c", subcore_axis_name="s")

    @functools.partial(
        pl.kernel,
        mesh=mesh,
        out_type=jax.ShapeDtypeStruct(
            (n_workers * iters * ROWS_PER_CHUNK, d), jnp.float32),
        scratch_types=[
            pltpu.VMEM((idx_per_tile,), jnp.int32),
            pltpu.VMEM((w_per_tile,), jnp.float32),
            pltpu.VMEM((2, IDX_PER_CHUNK, d2), jnp.int32),
            pltpu.VMEM((ROWS_PER_CHUNK, d), jnp.float32),
            pltpu.SemaphoreType.DMA,
            pltpu.SemaphoreType.DMA,
        ],
    )
    def k(xt_hbm, cols_hbm, vals_hbm, out_hbm, idx_v, w_v, gath_v, outc_v,
          gsem0, gsem1):
        wid = lax.axis_index("s") * num_cores + lax.axis_index("c")
        gsems = (gsem0, gsem1)

        pltpu.sync_copy(cols_hbm.at[pl.ds(wid * idx_per_tile, idx_per_tile)],
                        idx_v)
        pltpu.sync_copy(vals_hbm.at[pl.ds(wid * w_per_tile, w_per_tile)],
                        w_v)

        def gather_desc(i, p):
            return pltpu.make_async_copy(
                xt_hbm.at[idx_v.at[pl.ds(i * IDX_PER_CHUNK, IDX_PER_CHUNK)]],
                gath_v.at[p], gsems[p])

        def compute(i, p):
            gb = gath_v.at[p]

            def row_body(j, _):
                base = j * NNZ_PER_ROW
                w_vec = w_v[pl.ds(i * IDX_PER_CHUNK + base, LANES)]

                def col_body(v, _):
                    sl = pl.ds(v * LANES, LANES)
                    himask = jnp.int32(-65536)
                    g32 = gb[base, sl]
                    acc_a = w_vec[0] * lax.bitcast_convert_type(
                        g32 << 16, jnp.float32)
                    acc_b = w_vec[0] * lax.bitcast_convert_type(
                        g32 & himask, jnp.float32)
                    for kk in range(1, NNZ_PER_ROW):
                        g32 = gb[base + kk, sl]
                        acc_a = acc_a + w_vec[kk] * lax.bitcast_convert_type(
                            g32 << 16, jnp.float32)
                        acc_b = acc_b + w_vec[kk] * lax.bitcast_convert_type(
                            g32 & himask, jnp.float32)
                    outc_v[j, sl] = acc_a
                    outc_v[j, pl.ds(d2 + v * LANES, LANES)] = acc_b
                    return 0

                lax.fori_loop(0, d_groups, col_body, 0, unroll=2)
                return 0

            lax.fori_loop(0, ROWS_PER_CHUNK, row_body, 0)
            c = i * n_workers + wid
            pltpu.sync_copy(
                outc_v,
                out_hbm.at[pl.ds(c * ROWS_PER_CHUNK, ROWS_PER_CHUNK)])

        gather_desc(0, 0).start()
        gather_desc(1, 1).start()

        def chunk_pair(i2, _):
            for p in range(2):
                i = i2 + p
                gather_desc(i, p).wait()
                compute(i, p)

                @pl.when(i + 2 < iters)
                def _():
                    gather_desc(i + 2, p).start()
            return 0

        assert iters % 2 == 0
        lax.fori_loop(0, iters // 2, lambda h, a: chunk_pair(h * 2, a), 0)

    return k(xt, cols_t, vals_t)


def kernel(x, va_rows, va_cols, va_vals):
    b, ch, v_in = x.shape
    d = b * ch
    nnz = va_cols.shape[0]
    v_out = nnz // NNZ_PER_ROW

    info = plsc.get_sparse_core_info()
    n_workers = info.num_cores * info.num_subcores
    n_chunks = (v_out + ROWS_PER_CHUNK - 1) // ROWS_PER_CHUNK
    iters = (n_chunks + n_workers - 1) // n_workers
    iters = iters + (iters % 2)
    n_chunks_pad = iters * n_workers
    pad = n_chunks_pad * IDX_PER_CHUNK - nnz

    x16 = lax.optimization_barrier(x.reshape(d, v_in).astype(jnp.bfloat16))
    xt = jnp.transpose(x16)
    xt = lax.bitcast_convert_type(
        xt.reshape(v_in, d // 2, 2), jnp.int32)

    cols_p = jnp.concatenate([va_cols, jnp.zeros((pad,), jnp.int32)])
    vals_p = jnp.concatenate([va_vals, jnp.zeros((pad,), jnp.float32)])
    cols_t = jnp.transpose(
        cols_p.reshape(iters, n_workers, IDX_PER_CHUNK),
        (1, 0, 2)).reshape(n_workers * iters * IDX_PER_CHUNK)
    vals_t = jnp.pad(
        jnp.transpose(vals_p.reshape(iters, n_workers, IDX_PER_CHUNK),
                      (1, 0, 2)).reshape(n_workers, iters * IDX_PER_CHUNK),
        ((0, 0), (0, LANES))).reshape(-1)

    out_t = _sc_spmm(xt, cols_t, vals_t, iters, n_workers, info.num_cores, d)
    out_f = jnp.transpose(out_t[:v_out]).reshape(2, d // 2, v_out)
    out_f = lax.optimization_barrier(out_f)
    out_f = jnp.transpose(out_f, (1, 0, 2)).reshape(d, v_out)
    return out_f.reshape(b, ch, v_out)


# speedup vs baseline: 2.3434x; 2.3434x over previous
import functools

import jax
import jax.numpy as jnp
from jax import lax
from jax.experimental import pallas as pl
from jax.experimental.pallas import tpu as pltpu
from jax.experimental.pallas import tpu_sc as plsc

NNZ_PER_ROW = 7
ROWS_PER_CHUNK = 8
IDX_PER_CHUNK = NNZ_PER_ROW * ROWS_PER_CHUNK
LANES = 16


def _sc_spmm(xt, cols_t, vals_t, iters, n_workers, num_cores, d):
    d_groups = d // LANES
    idx_per_tile = iters * IDX_PER_CHUNK
    w_per_tile = idx_per_tile + LANES
    mesh = plsc.VectorSubcoreMesh(core_axis_name="c", subcore_axis_name="s")

    @functools.partial(
        pl.kernel,
        mesh=mesh,
        out_type=jax.ShapeDtypeStruct(
            (n_workers * iters * ROWS_PER_CHUNK, d), jnp.float32),
        scratch_types=[
            pltpu.VMEM((idx_per_tile,), jnp.int32),
            pltpu.VMEM((w_per_tile,), jnp.float32),
            pltpu.VMEM((2, IDX_PER_CHUNK, d), jnp.float32),
            pltpu.VMEM((ROWS_PER_CHUNK, d), jnp.float32),
            pltpu.SemaphoreType.DMA,
            pltpu.SemaphoreType.DMA,
        ],
    )
    def k(xt_hbm, cols_hbm, vals_hbm, out_hbm, idx_v, w_v, gath_v, outc_v,
          gsem0, gsem1):
        wid = lax.axis_index("s") * num_cores + lax.axis_index("c")
        gsems = (gsem0, gsem1)

        pltpu.sync_copy(cols_hbm.at[pl.ds(wid * idx_per_tile, idx_per_tile)],
                        idx_v)
        pltpu.sync_copy(vals_hbm.at[pl.ds(wid * w_per_tile, w_per_tile)],
                        w_v)

        def gather_desc(i, p):
            return pltpu.make_async_copy(
                xt_hbm.at[idx_v.at[pl.ds(i * IDX_PER_CHUNK, IDX_PER_CHUNK)]],
                gath_v.at[p], gsems[p])

        def compute(i, p):
            gb = gath_v.at[p]

            def row_body(j, _):
                base = j * NNZ_PER_ROW
                w_vec = w_v[pl.ds(i * IDX_PER_CHUNK + base, LANES)]

                def col_body(v, _):
                    sl = pl.ds(v * LANES, LANES)
                    acc = w_vec[0] * gb[base, sl]
                    for kk in range(1, NNZ_PER_ROW):
                        acc = acc + w_vec[kk] * gb[base + kk, sl]
                    outc_v[j, sl] = acc
                    return 0

                lax.fori_loop(0, d_groups, col_body, 0, unroll=2)
                return 0

            lax.fori_loop(0, ROWS_PER_CHUNK, row_body, 0)
            c = i * n_workers + wid
            pltpu.sync_copy(
                outc_v,
                out_hbm.at[pl.ds(c * ROWS_PER_CHUNK, ROWS_PER_CHUNK)])

        gather_desc(0, 0).start()
        gather_desc(1, 1).start()

        def chunk_pair(i2, _):
            for p in range(2):
                i = i2 + p
                gather_desc(i, p).wait()
                compute(i, p)

                @pl.when(i + 2 < iters)
                def _():
                    gather_desc(i + 2, p).start()
            return 0

        assert iters % 2 == 0
        lax.fori_loop(0, iters // 2, lambda h, a: chunk_pair(h * 2, a), 0)

    return k(xt, cols_t, vals_t)


def kernel(x, va_rows, va_cols, va_vals):
    b, ch, v_in = x.shape
    d = b * ch
    nnz = va_cols.shape[0]
    v_out = nnz // NNZ_PER_ROW

    info = plsc.get_sparse_core_info()
    n_workers = info.num_cores * info.num_subcores
    n_chunks = (v_out + ROWS_PER_CHUNK - 1) // ROWS_PER_CHUNK
    iters = (n_chunks + n_workers - 1) // n_workers
    iters = iters + (iters % 2)
    n_chunks_pad = iters * n_workers
    pad = n_chunks_pad * IDX_PER_CHUNK - nnz

    xt = jnp.transpose(x.reshape(d, v_in))

    cols_p = jnp.concatenate([va_cols, jnp.zeros((pad,), jnp.int32)])
    vals_p = jnp.concatenate([va_vals, jnp.zeros((pad,), jnp.float32)])
    cols_t = jnp.transpose(
        cols_p.reshape(iters, n_workers, IDX_PER_CHUNK),
        (1, 0, 2)).reshape(n_workers * iters * IDX_PER_CHUNK)
    vals_t = jnp.pad(
        jnp.transpose(vals_p.reshape(iters, n_workers, IDX_PER_CHUNK),
                      (1, 0, 2)).reshape(n_workers, iters * IDX_PER_CHUNK),
        ((0, 0), (0, LANES))).reshape(-1)

    out_t = _sc_spmm(xt, cols_t, vals_t, iters, n_workers, info.num_cores, d)
    return jnp.transpose(out_t[:v_out]).reshape(b, ch, v_out)
